# baseline (speedup 1.0000x reference)
import functools

import jax
import jax.numpy as jnp
from jax import lax
from jax.experimental import pallas as pl
from jax.experimental.pallas import tpu as pltpu
from jax.experimental.pallas import tpu_sc as plsc

_N_FIELDS = 26
_VOCAB = 100000
_DIM = 64
_BATCH = 4096
_BAG = 20
_NW = 32
_BAGS_PER_W = _BATCH // _NW
_CHUNK_BAGS = 32
_N_CHUNKS = _BAGS_PER_W // _CHUNK_BAGS
_ROWS_PER_CHUNK = _CHUNK_BAGS * _BAG
_SLICES = _ROWS_PER_CHUNK // 128
_IDX_ROWS = _BATCH * _BAG // 128


def _make_field_kernel():
    mesh = plsc.VectorSubcoreMesh(core_axis_name="c", subcore_axis_name="s")

    @functools.partial(
        pl.kernel,
        mesh=mesh,
        out_type=jax.ShapeDtypeStruct((_BATCH // 2, 2 * _DIM), jnp.float32),
        compiler_params=pltpu.CompilerParams(use_tc_tiling_on_sc=False),
        scratch_types=[
            pltpu.VMEM((_SLICES, 128), jnp.int32),
            pltpu.VMEM((_SLICES, 128), jnp.int32),
            pltpu.VMEM((_ROWS_PER_CHUNK, _DIM), jnp.float32),
            pltpu.VMEM((_ROWS_PER_CHUNK, _DIM), jnp.float32),
            pltpu.VMEM((_CHUNK_BAGS // 2, 2 * _DIM), jnp.float32),
            pltpu.VMEM((_CHUNK_BAGS // 2, 2 * _DIM), jnp.float32),
            pltpu.SemaphoreType.DMA,
            pltpu.SemaphoreType.DMA,
            pltpu.SemaphoreType.DMA,
            pltpu.SemaphoreType.DMA,
            pltpu.SemaphoreType.DMA,
            pltpu.SemaphoreType.DMA,
        ],
    )
    def k(idx_hbm, w_hbm, out_hbm,
          ib0, ib1, rb0, rb1, ob0, ob1,
          isem0, isem1, gsem0, gsem1, osem0, osem1):
        wid = lax.axis_index("s") * 2 + lax.axis_index("c")
        ibs = (ib0, ib1)
        rbs = (rb0, rb1)
        obs = (ob0, ob1)
        isems = (isem0, isem1)
        gsems = (gsem0, gsem1)
        osems = (osem0, osem1)

        def fire_idx(c, p):
            row0 = wid * (_SLICES * _N_CHUNKS) + c * _SLICES
            pltpu.async_copy(
                idx_hbm.at[pl.ds(row0, _SLICES)], ibs[p], isems[p])

        def drain_idx(p):
            pltpu.make_async_copy(
                idx_hbm.at[pl.ds(0, _SLICES)], ibs[p], isems[p]).wait()

        def fire_gathers(p):
            for i in range(_SLICES):
                pltpu.async_copy(
                    w_hbm.at[ibs[p].at[i]],
                    rbs[p].at[pl.ds(i * 128, 128)],
                    gsems[p],
                )

        def drain_gathers(p):
            pltpu.make_async_copy(
                w_hbm.at[pl.ds(0, _ROWS_PER_CHUNK)], rbs[p], gsems[p]
            ).wait()

        def fire_out(c, p):
            base = (wid * _BAGS_PER_W + c * _CHUNK_BAGS) // 2
            pltpu.async_copy(
                obs[p], out_hbm.at[pl.ds(base, _CHUNK_BAGS // 2)], osems[p]
            )

        def drain_out(p):
            pltpu.make_async_copy(
                obs[p], out_hbm.at[pl.ds(0, _CHUNK_BAGS // 2)], osems[p]
            ).wait()

        def reduce_chunk(p):
            rows = rbs[p]
            outb = obs[p]

            def body(b, carry):
                base = b * _BAG
                half = (b % 2) * _DIM
                accs = [rows[base, pl.ds(tt * 16, 16)]
                        for tt in range(_DIM // 16)]
                for j in range(1, _BAG):
                    for tt in range(_DIM // 16):
                        accs[tt] = accs[tt] + rows[base + j,
                                                   pl.ds(tt * 16, 16)]
                for tt in range(_DIM // 16):
                    outb[b // 2, pl.ds(half + tt * 16, 16)] = accs[tt]
                return carry

            lax.fori_loop(0, _CHUNK_BAGS, body, 0, unroll=False)

        fire_idx(0, 0)
        fire_idx(1, 1)
        drain_idx(0)
        fire_gathers(0)
        for c in range(_N_CHUNKS):
            p = c % 2
            drain_gathers(p)
            if c + 2 < _N_CHUNKS:
                fire_idx(c + 2, p)
            if c + 1 < _N_CHUNKS:
                drain_idx(1 - p)
                fire_gathers(1 - p)
            if c >= 2:
                drain_out(p)
            reduce_chunk(p)
            fire_out(c, p)
        drain_out(0)
        drain_out(1)

    return k


def kernel(indices, offsets, W):
    del offsets
    idxr = indices.reshape(_N_FIELDS, _IDX_ROWS, 128)
    field_kernel = _make_field_kernel()
    outs = [field_kernel(idxr[k], W[k]) for k in range(_N_FIELDS)]
    out = jnp.stack(outs)
    return out.reshape(_N_FIELDS, _BATCH, _DIM)
